# baseline (speedup 1.0000x reference)
import functools

import jax
import jax.numpy as jnp
from jax import lax
from jax.experimental import pallas as pl
from jax.experimental.pallas import tpu as pltpu
from jax.experimental.pallas import tpu_sc as plsc

MS = 1000000
C = 20000
CHUNKS = MS // C
NTASK = 5 * CHUNKS
NW = 32


def _setup_consts():
    key = jax.random.key(1)
    k_ints, k_shuf = jax.random.split(key)
    ints = jax.random.randint(k_ints, (5,), 0, MS)
    keys = jax.random.split(k_shuf, 5)
    p = jax.vmap(lambda k: jax.random.permutation(k, MS))(keys)
    return (p - ints[:, None]).astype(jnp.int32)


_D = _setup_consts()


def _task(x_hbm, d_hbm, o_hbm, xv, dv, mv, t):
    j = t // CHUNKS
    c0 = (t % CHUNKS) * C
    pltpu.sync_copy(x_hbm.at[j, pl.ds(c0, C)], xv)
    pltpu.sync_copy(d_hbm.at[j, pl.ds(c0, C)], dv)

    def step(i, _):
        s = pl.ds(i * 16, 16)
        ones = jnp.where(dv[s] < 0, 1.0, 0.0)
        xv[s] = xv[s] * ones
        mv[s] = ones
        return 0

    lax.fori_loop(0, C // 16, step, 0)
    pltpu.sync_copy(xv, o_hbm.at[j, pl.ds(c0, C)])
    pltpu.sync_copy(mv, o_hbm.at[j, pl.ds(MS + c0, C)])


def _sc_kernel(x_hbm, d_hbm, o_hbm, xv, dv, mv):
    w = lax.axis_index("s") * 2 + lax.axis_index("c")
    nt = (NTASK - w + NW - 1) // NW

    def body(i, _):
        _task(x_hbm, d_hbm, o_hbm, xv, dv, mv, w + i * NW)
        return 0

    lax.fori_loop(0, nt, body, 0)


def kernel(x):
    mesh = plsc.VectorSubcoreMesh(
        core_axis_name="c", subcore_axis_name="s", num_cores=2, num_subcores=16
    )
    run = functools.partial(
        pl.kernel,
        mesh=mesh,
        out_type=jax.ShapeDtypeStruct((5, 2 * MS), jnp.float32),
        scratch_types=[
            pltpu.VMEM((C,), jnp.float32),
            pltpu.VMEM((C,), jnp.int32),
            pltpu.VMEM((C,), jnp.float32),
        ],
        compiler_params=pltpu.CompilerParams(use_tc_tiling_on_sc=False),
    )(_sc_kernel)
    return run(x, _D)
